# baseline (speedup 1.0000x reference)
import jax
import jax.numpy as jnp
from jax import lax
from jax.experimental import pallas as pl
from jax.experimental.pallas import tpu as pltpu
from jax.experimental.pallas import tpu_sc as plsc

B, D = 16384, 128
NT = 1000
_NC, _NS, _L = 2, 16, 16
NW = _NC * _NS
BPW = B // NW
CH = 64
NCH = BPW // CH
U = 4


def _body(xs_hbm, xn_hbm, ts_hbm, t1_hbm, t2_hbm, out_hbm,
          ts_v, t1_v, t2_v, s1_v, s2_v,
          xs0_v, xs1_v, xn0_v, xn1_v, o0_v, o1_v,
          ld0, ld1, st0, st1, pr):
    wid = lax.axis_index("s") * _NC + lax.axis_index("c")
    base = wid * BPW

    xs_b, xn_b, o_b = [xs0_v, xs1_v], [xn0_v, xn1_v], [o0_v, o1_v]
    lds, sts = [ld0, ld1], [st0, st1]
    ld_h, st_h = {}, {}

    def issue_load(c):
        bi = c % 2
        row0 = base + c * CH
        ld_h[c] = (
            pltpu.async_copy(xs_hbm.at[pl.ds(row0, CH)], xs_b[bi], lds[bi]),
            pltpu.async_copy(xn_hbm.at[pl.ds(row0, CH)], xn_b[bi], lds[bi]),
        )

    issue_load(0)
    issue_load(1)
    pro = (
        pltpu.async_copy(ts_hbm.at[pl.ds(base, BPW)], ts_v, pr),
        pltpu.async_copy(t1_hbm, t1_v, pr),
        pltpu.async_copy(t2_hbm, t2_v, pr),
    )
    for h in pro:
        h.wait()

    @plsc.parallel_loop(0, BPW // _L, step=1, unroll=4)
    def _coef(i):
        sl = pl.ds(i * _L, _L)
        idx = ts_v[sl]
        s1_v[sl] = plsc.load_gather(t1_v, [idx])
        s2_v[sl] = plsc.load_gather(t2_v, [idx])

    def compute(c):
        bi = c % 2
        xs_r, xn_r, o_r = xs_b[bi], xn_b[bi], o_b[bi]
        loc0 = c * CH

        @plsc.parallel_loop(0, CH, step=1, unroll=U)
        def _row(r):
            bidx = jnp.full((_L,), loc0 + r, dtype=jnp.int32)
            s1 = plsc.load_gather(s1_v, [bidx])
            s2 = plsc.load_gather(s2_v, [bidx])
            for j in range(D // _L):
                sl = pl.ds(j * _L, _L)
                o_r[r, sl] = s1 * xs_r[r, sl] + s2 * xn_r[r, sl]

    for c in range(NCH):
        if 1 < c + 1 < NCH:
            issue_load(c + 1)
        for h in ld_h.pop(c):
            h.wait()
        if c >= 2:
            st_h.pop(c - 2).wait()
        compute(c)
        st_h[c] = pltpu.async_copy(
            o_b[c % 2], out_hbm.at[pl.ds(base + c * CH, CH)], sts[c % 2])
    st_h.pop(NCH - 2).wait()
    st_h.pop(NCH - 1).wait()


def kernel(x_start, x_noise, timesteps,
           sqrt_alphas_cumprod, sqrt_one_minus_alphas_cumprod):
    run = pl.kernel(
        _body,
        mesh=plsc.VectorSubcoreMesh(core_axis_name="c", subcore_axis_name="s"),
        out_type=jax.ShapeDtypeStruct((B, D), jnp.float32),
        scratch_types=[
            pltpu.VMEM((BPW,), jnp.int32),
            pltpu.VMEM((NT,), jnp.float32),
            pltpu.VMEM((NT,), jnp.float32),
            pltpu.VMEM((BPW,), jnp.float32),
            pltpu.VMEM((BPW,), jnp.float32),
            pltpu.VMEM((CH, D), jnp.float32),
            pltpu.VMEM((CH, D), jnp.float32),
            pltpu.VMEM((CH, D), jnp.float32),
            pltpu.VMEM((CH, D), jnp.float32),
            pltpu.VMEM((CH, D), jnp.float32),
            pltpu.VMEM((CH, D), jnp.float32),
            pltpu.SemaphoreType.DMA,
            pltpu.SemaphoreType.DMA,
            pltpu.SemaphoreType.DMA,
            pltpu.SemaphoreType.DMA,
            pltpu.SemaphoreType.DMA,
        ],
        compiler_params=pltpu.CompilerParams(
            needs_layout_passes=False, skip_device_barrier=True),
    )
    return run(x_start, x_noise, timesteps,
               sqrt_alphas_cumprod, sqrt_one_minus_alphas_cumprod)


# speedup vs baseline: 1.0899x; 1.0899x over previous
import jax
import jax.numpy as jnp
from jax import lax
from jax.experimental import pallas as pl
from jax.experimental.pallas import tpu as pltpu
from jax.experimental.pallas import tpu_sc as plsc

B, D = 16384, 128
NT = 1000
_NC, _NS, _L = 2, 16, 16
NW = _NC * _NS
BPW = B // NW
CH = 128
NCH = BPW // CH
U = 4


def _body(xs_hbm, xn_hbm, ts_hbm, t1_hbm, t2_hbm, out_hbm,
          ts_v, t1_v, t2_v, s1_v, s2_v,
          xs0_v, xs1_v, xn0_v, xn1_v, o0_v, o1_v,
          ld0, ld1, st0, st1, pr):
    wid = lax.axis_index("s") * _NC + lax.axis_index("c")
    base = wid * BPW

    xs_b, xn_b, o_b = [xs0_v, xs1_v], [xn0_v, xn1_v], [o0_v, o1_v]
    lds, sts = [ld0, ld1], [st0, st1]
    ld_h, st_h = {}, {}

    def issue_load(c):
        bi = c % 2
        row0 = base + c * CH
        ld_h[c] = (
            pltpu.async_copy(xs_hbm.at[pl.ds(row0, CH)], xs_b[bi], lds[bi]),
            pltpu.async_copy(xn_hbm.at[pl.ds(row0, CH)], xn_b[bi], lds[bi]),
        )

    issue_load(0)
    issue_load(1)
    pro = (
        pltpu.async_copy(ts_hbm.at[pl.ds(base, BPW)], ts_v, pr),
        pltpu.async_copy(t1_hbm, t1_v, pr),
        pltpu.async_copy(t2_hbm, t2_v, pr),
    )
    for h in pro:
        h.wait()

    @plsc.parallel_loop(0, BPW // _L, step=1, unroll=4)
    def _coef(i):
        sl = pl.ds(i * _L, _L)
        idx = ts_v[sl]
        s1_v[sl] = plsc.load_gather(t1_v, [idx])
        s2_v[sl] = plsc.load_gather(t2_v, [idx])

    def compute(c):
        bi = c % 2
        xs_r, xn_r, o_r = xs_b[bi], xn_b[bi], o_b[bi]
        loc0 = c * CH

        @plsc.parallel_loop(0, CH, step=1, unroll=U)
        def _row(r):
            bidx = jnp.full((_L,), loc0 + r, dtype=jnp.int32)
            s1 = plsc.load_gather(s1_v, [bidx])
            s2 = plsc.load_gather(s2_v, [bidx])
            for j in range(D // _L):
                sl = pl.ds(j * _L, _L)
                o_r[r, sl] = s1 * xs_r[r, sl] + s2 * xn_r[r, sl]

    for c in range(NCH):
        if 1 < c + 1 < NCH:
            issue_load(c + 1)
        for h in ld_h.pop(c):
            h.wait()
        if c >= 2:
            st_h.pop(c - 2).wait()
        compute(c)
        st_h[c] = pltpu.async_copy(
            o_b[c % 2], out_hbm.at[pl.ds(base + c * CH, CH)], sts[c % 2])
    st_h.pop(NCH - 2).wait()
    st_h.pop(NCH - 1).wait()


def kernel(x_start, x_noise, timesteps,
           sqrt_alphas_cumprod, sqrt_one_minus_alphas_cumprod):
    run = pl.kernel(
        _body,
        mesh=plsc.VectorSubcoreMesh(core_axis_name="c", subcore_axis_name="s"),
        out_type=jax.ShapeDtypeStruct((B, D), jnp.float32),
        scratch_types=[
            pltpu.VMEM((BPW,), jnp.int32),
            pltpu.VMEM((NT,), jnp.float32),
            pltpu.VMEM((NT,), jnp.float32),
            pltpu.VMEM((BPW,), jnp.float32),
            pltpu.VMEM((BPW,), jnp.float32),
            pltpu.VMEM((CH, D), jnp.float32),
            pltpu.VMEM((CH, D), jnp.float32),
            pltpu.VMEM((CH, D), jnp.float32),
            pltpu.VMEM((CH, D), jnp.float32),
            pltpu.VMEM((CH, D), jnp.float32),
            pltpu.VMEM((CH, D), jnp.float32),
            pltpu.SemaphoreType.DMA,
            pltpu.SemaphoreType.DMA,
            pltpu.SemaphoreType.DMA,
            pltpu.SemaphoreType.DMA,
            pltpu.SemaphoreType.DMA,
        ],
        compiler_params=pltpu.CompilerParams(
            needs_layout_passes=False, skip_device_barrier=True),
    )
    return run(x_start, x_noise, timesteps,
               sqrt_alphas_cumprod, sqrt_one_minus_alphas_cumprod)
